# baseline (speedup 1.0000x reference)
import functools

import jax
import jax.numpy as jnp
from jax import lax
from jax.experimental import pallas as pl
from jax.experimental.pallas import tpu as pltpu
from jax.experimental.pallas import tpu_sc as plsc

_N = 4096
_BR = 512
_FS = _BR * _N
_FR = _FS // 128

_ROT = ((13, 15, 26, 6), (17, 29, 16, 24))


def _threefry_bits(cnt):
    ks = (jnp.uint32(0), jnp.uint32(42), jnp.uint32(0 ^ 42 ^ 0x1BD11BDA))
    x0 = jnp.full(cnt.shape, ks[0], jnp.uint32)
    x1 = cnt + ks[1]
    for i in range(5):
        for r in _ROT[i % 2]:
            x0 = x0 + x1
            x1 = (x1 << jnp.uint32(r)) | (x1 >> jnp.uint32(32 - r))
            x1 = x1 ^ x0
        x0 = x0 + ks[(i + 1) % 3]
        x1 = x1 + ks[(i + 2) % 3] + jnp.uint32(i + 1)
    return x0 ^ x1


_SCW = 32
_WPW = (_N * _N) // _SCW
_CHW = 16384
_NCH = _WPW // _CHW


def _make_ei_sc():
    mesh = plsc.VectorSubcoreMesh(core_axis_name="c", subcore_axis_name="s")

    @functools.partial(
        pl.kernel,
        mesh=mesh,
        out_type=jax.ShapeDtypeStruct((2, _N * _N), jnp.int32),
        scratch_types=[
            pltpu.VMEM((_CHW,), jnp.int32),
            pltpu.VMEM((_CHW,), jnp.int32),
            pltpu.VMEM((_CHW,), jnp.int32),
            pltpu.SemaphoreType.DMA,
            pltpu.SemaphoreType.DMA,
        ],
    )
    def _ei_sc(out_hbm, e1_v, e0a_v, e0b_v, sem1, sem0):
        wid = lax.axis_index("s") * 2 + lax.axis_index("c")
        f0 = wid * _WPW
        lanes = lax.iota(jnp.int32, 16)

        def fill1(t, _):
            e1_v[pl.ds(16 * t, 16)] = ((16 * t) & (_N - 1)) + lanes
            return 0

        lax.fori_loop(0, _CHW // 16, fill1, 0, unroll=4)
        d1 = []
        for ch in range(_NCH):
            d1.append(pltpu.async_copy(
                e1_v, out_hbm.at[1, pl.ds(f0 + ch * _CHW, _CHW)], sem1))

        def fill0(buf, ch):
            n0 = (f0 + ch * _CHW) >> 12

            def body(t, _):
                buf[pl.ds(16 * t, 16)] = jnp.full((16,), n0 + (t >> 8), jnp.int32)
                return 0

            lax.fori_loop(0, _CHW // 16, body, 0, unroll=4)

        d0 = []
        for ch in range(_NCH):
            buf = e0a_v if ch % 2 == 0 else e0b_v
            if ch >= 2:
                d0[ch - 2].wait()
            fill0(buf, ch)
            d0.append(pltpu.async_copy(
                buf, out_hbm.at[0, pl.ds(f0 + ch * _CHW, _CHW)], sem0))
        for d in d0[-2:]:
            d.wait()
        for d in d1:
            d.wait()

    return _ei_sc


def _samples_body(scale_ref, scores_ref, samples_ref):
    i = pl.program_id(0)
    row = jax.lax.broadcasted_iota(jnp.int32, (_BR, _N), 0) + i * _BR
    col = jax.lax.broadcasted_iota(jnp.int32, (_BR, _N), 1)
    cnt = (row * _N + col).astype(jnp.uint32)
    bits = _threefry_bits(cnt)
    mant = (bits >> jnp.uint32(9)) | jnp.uint32(0x3F800000)
    u = jax.lax.bitcast_convert_type(mant, jnp.float32) - jnp.float32(1.0)
    p = jax.nn.sigmoid(scores_ref[...])
    scale = scale_ref[0, 0]
    vals = jnp.where(u < p, scale, jnp.float32(0.0))
    samples_ref[...] = vals.reshape(_FR, 128)


def kernel(x, n_adjs, scores):
    del x
    scale = jnp.asarray(n_adjs, jnp.float32).reshape(1, 1)
    ei2 = _make_ei_sc()()
    samples = pl.pallas_call(
        _samples_body,
        grid=(_N // _BR,),
        in_specs=[
            pl.BlockSpec(memory_space=pltpu.SMEM),
            pl.BlockSpec((_BR, _N), lambda i: (i, 0)),
        ],
        out_specs=pl.BlockSpec((_FR, 128), lambda i: (i, 0)),
        out_shape=jax.ShapeDtypeStruct((_N * _N // 128, 128), jnp.float32),
        compiler_params=pltpu.CompilerParams(
            dimension_semantics=("arbitrary",),
        ),
    )(scale, scores)
    return (ei2, samples.reshape(_N * _N))


# speedup vs baseline: 1.3161x; 1.3161x over previous
import functools

import jax
import jax.numpy as jnp
from jax import lax
from jax.experimental import pallas as pl
from jax.experimental.pallas import tpu as pltpu
from jax.experimental.pallas import tpu_sc as plsc

_N = 4096
_BR = 128
_FS = _BR * _N
_FR = _FS // 128

_ROT = ((13, 15, 26, 6), (17, 29, 16, 24))


def _threefry_bits(cnt):
    ks = (jnp.uint32(0), jnp.uint32(42), jnp.uint32(0 ^ 42 ^ 0x1BD11BDA))
    x0 = jnp.full(cnt.shape, ks[0], jnp.uint32)
    x1 = cnt + ks[1]
    for i in range(5):
        for r in _ROT[i % 2]:
            x0 = x0 + x1
            x1 = (x1 << jnp.uint32(r)) | (x1 >> jnp.uint32(32 - r))
            x1 = x1 ^ x0
        x0 = x0 + ks[(i + 1) % 3]
        x1 = x1 + ks[(i + 2) % 3] + jnp.uint32(i + 1)
    return x0 ^ x1


_SCW = 32
_WPW = (_N * _N) // _SCW
_CHW = 16384
_NCH = _WPW // _CHW


def _make_ei_sc():
    mesh = plsc.VectorSubcoreMesh(core_axis_name="c", subcore_axis_name="s")

    @functools.partial(
        pl.kernel,
        mesh=mesh,
        out_type=jax.ShapeDtypeStruct((2, _N * _N), jnp.int32),
        scratch_types=[
            pltpu.VMEM((_CHW,), jnp.int32),
            pltpu.VMEM((_CHW,), jnp.int32),
            pltpu.VMEM((_CHW,), jnp.int32),
            pltpu.SemaphoreType.DMA,
            pltpu.SemaphoreType.DMA,
        ],
    )
    def _ei_sc(out_hbm, e1_v, e0a_v, e0b_v, sem1, sem0):
        wid = lax.axis_index("s") * 2 + lax.axis_index("c")
        f0 = wid * _WPW
        lanes = lax.iota(jnp.int32, 16)

        def fill1(t, _):
            e1_v[pl.ds(16 * t, 16)] = ((16 * t) & (_N - 1)) + lanes
            return 0

        lax.fori_loop(0, _CHW // 16, fill1, 0, unroll=4)
        d1 = []
        for ch in range(_NCH):
            d1.append(pltpu.async_copy(
                e1_v, out_hbm.at[1, pl.ds(f0 + ch * _CHW, _CHW)], sem1))

        def fill0(buf, ch):
            n0 = (f0 + ch * _CHW) >> 12

            def body(t, _):
                buf[pl.ds(16 * t, 16)] = jnp.full((16,), n0 + (t >> 8), jnp.int32)
                return 0

            lax.fori_loop(0, _CHW // 16, body, 0, unroll=4)

        d0 = []
        for ch in range(_NCH):
            buf = e0a_v if ch % 2 == 0 else e0b_v
            if ch >= 2:
                d0[ch - 2].wait()
            fill0(buf, ch)
            d0.append(pltpu.async_copy(
                buf, out_hbm.at[0, pl.ds(f0 + ch * _CHW, _CHW)], sem0))
        for d in d0[-2:]:
            d.wait()
        for d in d1:
            d.wait()

    return _ei_sc


def _samples_body(scale_ref, scores_ref, samples_ref):
    i = pl.program_id(0)
    row = jax.lax.broadcasted_iota(jnp.int32, (_BR, _N), 0) + i * _BR
    col = jax.lax.broadcasted_iota(jnp.int32, (_BR, _N), 1)
    cnt = (row * _N + col).astype(jnp.uint32)
    bits = _threefry_bits(cnt)
    mant = (bits >> jnp.uint32(9)) | jnp.uint32(0x3F800000)
    u = jax.lax.bitcast_convert_type(mant, jnp.float32) - jnp.float32(1.0)
    p = jax.nn.sigmoid(scores_ref[...])
    scale = scale_ref[0, 0]
    vals = jnp.where(u < p, scale, jnp.float32(0.0))
    samples_ref[...] = vals.reshape(_FR, 128)


def kernel(x, n_adjs, scores):
    del x
    scale = jnp.asarray(n_adjs, jnp.float32).reshape(1, 1)
    ei2 = _make_ei_sc()()
    samples = pl.pallas_call(
        _samples_body,
        grid=(_N // _BR,),
        in_specs=[
            pl.BlockSpec(memory_space=pltpu.SMEM),
            pl.BlockSpec((_BR, _N), lambda i: (i, 0)),
        ],
        out_specs=pl.BlockSpec((_FR, 128), lambda i: (i, 0)),
        out_shape=jax.ShapeDtypeStruct((_N * _N // 128, 128), jnp.float32),
        compiler_params=pltpu.CompilerParams(
            dimension_semantics=("arbitrary",),
        ),
    )(scale, scores)
    return (ei2, samples.reshape(_N * _N))
